# baseline (speedup 1.0000x reference)
import functools

import jax
import jax.numpy as jnp
from jax import lax
from jax.experimental import pallas as pl
from jax.experimental.pallas import tpu as pltpu
from jax.experimental.pallas import tpu_sc as plsc

N = 10000
E = 320000
NC, NS = 2, 16
EPT = E // NS
C = 80
NCHUNK = EPT // C
RB = 624
REM = N - RB * NS
RZ = RB + REM
F1, F2, F3 = 128, 64, 64
H = F2 // 2
CW = 16

_SC_PARAMS = pltpu.CompilerParams(use_tc_tiling_on_sc=False)


def _mesh():
    return plsc.VectorSubcoreMesh(core_axis_name="c", subcore_axis_name="s")


def _zero_acc(sid, z_h, acc):
    pltpu.sync_copy(z_h.at[pl.ds(0, RB)], acc.at[pl.ds(sid * RB, RB)])

    @pl.when(sid == 0)
    def _():
        pltpu.sync_copy(z_h.at[pl.ds(0, REM)], acc.at[pl.ds(RB * NS, REM)])


def _copy_out(sid, acc, out2d):
    rows = pl.ds(sid * RB, RB)
    pltpu.sync_copy(acc.at[rows], out2d.at[rows])

    @pl.when(sid == 0)
    def _():
        tail = pl.ds(RB * NS, REM)
        pltpu.sync_copy(acc.at[tail], out2d.at[tail])


def _sc_degrees(e3, ones_h, zeros_h):
    @functools.partial(
        pl.kernel,
        out_type=jax.ShapeDtypeStruct((NC, N, CW), jnp.float32),
        mesh=_mesh(),
        compiler_params=_SC_PARAMS,
        scratch_types=[
            pltpu.VMEM((EPT,), jnp.int32),
            pltpu.VMEM((C, CW), jnp.float32),
            pltpu.VMEM_SHARED((N, CW), jnp.float32),
            pltpu.SemaphoreType.DMA,
        ],
    )
    def k(e_h, ones_hr, z_h, deg_h, idx_v, ones_v, acc_s, ssem):
        cid = lax.axis_index("c")
        sid = lax.axis_index("s")
        _zero_acc(sid, z_h, acc_s)
        pltpu.sync_copy(ones_hr, ones_v)
        pltpu.sync_copy(e_h.at[cid].at[sid], idx_v)
        plsc.subcore_barrier()

        @pl.loop(0, NCHUNK, step=5)
        def _(j):
            for o in range(5):
                pltpu.async_copy(
                    ones_v, acc_s.at[idx_v.at[pl.ds((j + o) * C, C)]], ssem,
                    add=True)
            for o in range(5):
                pltpu.make_async_copy(
                    ones_v, acc_s.at[idx_v.at[pl.ds(0, C)]], ssem).wait()

        plsc.subcore_barrier()
        _copy_out(sid, acc_s, deg_h.at[cid])

    return k(e3, ones_h, zeros_h)


def _sc_propagate(tables, e3, zeros_list, Ds, K, swaps, splits):
    NT = len(tables)
    NB = NCHUNK // K
    assert NCHUNK % K == 0

    bufs_types = [pltpu.VMEM((C, D), jnp.float32)
                  for D in Ds for _k in range(K)]
    acc_types = [pltpu.VMEM_SHARED((N, D), jnp.float32) for D in Ds]
    sem_types = [pltpu.SemaphoreType.DMA] * (2 * NT)

    @functools.partial(
        pl.kernel,
        out_type=tuple(jax.ShapeDtypeStruct((NC, N, D), jnp.float32) for D in Ds),
        mesh=_mesh(),
        compiler_params=_SC_PARAMS,
        scratch_types=[
            pltpu.VMEM((EPT,), jnp.int32),
            pltpu.VMEM((EPT,), jnp.int32),
        ] + bufs_types + acc_types + sem_types,
    )
    def k(*refs):
        t_h = refs[:NT]
        e_h = refs[NT]
        z_h = refs[NT + 1:NT + 1 + NT]
        agg_h = refs[NT + 1 + NT:NT + 1 + 2 * NT]
        src_v, dst_v = refs[3 * NT + 1], refs[3 * NT + 2]
        p = 3 * NT + 3
        bufs = [[refs[p + i * K + k_] for k_ in range(K)] for i in range(NT)]
        p += NT * K
        accs = refs[p:p + NT]
        p += NT
        gsem = refs[p:p + NT]
        ssem = refs[p + NT:p + 2 * NT]

        cid = lax.axis_index("c")
        sid = lax.axis_index("s")
        for i in range(NT):
            _zero_acc(sid, z_h[i], accs[i])
        pltpu.sync_copy(e_h.at[0].at[sid], src_v)
        pltpu.sync_copy(e_h.at[1].at[sid], dst_v)
        plsc.subcore_barrier()

        def table(i):
            return t_h[i].at[cid] if splits[i] else t_h[i]

        def gidx(i, j):
            v = dst_v if swaps[i] else src_v
            return v.at[pl.ds(j * C, C)]

        def sidx(i, j):
            v = src_v if swaps[i] else dst_v
            return v.at[pl.ds(j * C, C)]

        @pl.loop(0, NB)
        def _(b):
            base = b * K
            for k_ in range(K):
                for i in range(NT):
                    pltpu.async_copy(table(i).at[gidx(i, base + k_)],
                                     bufs[i][k_], gsem[i])
            for k_ in range(K):
                for i in range(NT):
                    pltpu.make_async_copy(table(i).at[gidx(i, 0)],
                                          bufs[i][0], gsem[i]).wait()
            for k_ in range(K):
                for i in range(NT):
                    pltpu.async_copy(bufs[i][k_],
                                     accs[i].at[sidx(i, base + k_)],
                                     ssem[i], add=True)
            for k_ in range(K):
                for i in range(NT):
                    pltpu.make_async_copy(bufs[i][0],
                                          accs[i].at[sidx(i, 0)],
                                          ssem[i]).wait()

        plsc.subcore_barrier()
        for i in range(NT):
            _copy_out(sid, accs[i], agg_h[i].at[cid])

    return k(*tables, e3, *zeros_list)


def _sc_layer1(t1pair, e3, z64):
    return _sc_propagate([t1pair], e3, [z64], [F2], 5, [False], [True])[0]


def _sc_cpass(tnd, e3, z16):
    return _sc_propagate([tnd], e3, [z16], [CW], 5, [True], [False])[0]


def _sc_layer2(t2pair, e3, z32):
    return _sc_propagate([t2pair], e3, [z32], [H], 5, [False], [True])[0]


G = 10
BR = N // G


def _bs(shape, im):
    return pl.BlockSpec(shape, im)


def _row(i):
    return (i, 0)


def _prow(i):
    return (0, i, 0)


def _full(i):
    return (0, 0)


def _dot(a, b):
    ah = a.astype(jnp.bfloat16)
    al = (a - ah.astype(jnp.float32)).astype(jnp.bfloat16)
    bh = b.astype(jnp.bfloat16)
    bl = (b - bh.astype(jnp.float32)).astype(jnp.bfloat16)

    def d(u, v):
        return lax.dot_general(u, v, (((1,), (0,)), ((), ())),
                               preferred_element_type=jnp.float32)

    return d(ah, bh) + d(ah, bl) + d(al, bh)


def _tc_prep(x, W1, deg):
    def body(x_ref, w_ref, dg_ref, t1p_ref, nrm_ref):
        dego = dg_ref[0, :, 0:1]
        degi = dg_ref[1, :, 0:1]
        ns = jnp.where(dego > 0, lax.rsqrt(dego), 0.0)
        nd = jnp.where(degi > 0, lax.rsqrt(degi), 0.0)
        xs = x_ref[...] * ns
        t1p_ref[0] = _dot(xs, w_ref[:, :F2])
        t1p_ref[1] = _dot(xs, w_ref[:, F2:])
        nrm_ref[0] = jnp.broadcast_to(ns, (BR, CW))
        nrm_ref[1] = jnp.broadcast_to(nd, (BR, CW))

    return pl.pallas_call(
        body,
        grid=(G,),
        in_specs=[_bs((BR, F1), _row), _bs((F1, F1), _full),
                  _bs((NC, BR, CW), _prow)],
        out_specs=(_bs((NC, BR, F2), _prow), _bs((NC, BR, CW), _prow)),
        out_shape=(jax.ShapeDtypeStruct((NC, N, F2), jnp.float32),
                   jax.ShapeDtypeStruct((NC, N, CW), jnp.float32)),
    )(x, W1, deg)


def _tc_mid(agg1, nrm, b1r, W2):
    def body(a_ref, n_ref, b_ref, w_ref, t2p_ref):
        ns = n_ref[0, :, 0:1]
        nd = n_ref[1, :, 0:1]
        h1a = jnp.maximum(a_ref[0] * nd + b_ref[:, :F2], 0.0) * ns
        h1b = jnp.maximum(a_ref[1] * nd + b_ref[:, F2:], 0.0) * ns
        t2p_ref[0] = _dot(h1a, w_ref[:F2, :H]) + _dot(h1b, w_ref[F2:, :H])
        t2p_ref[1] = _dot(h1a, w_ref[:F2, H:]) + _dot(h1b, w_ref[F2:, H:])

    return pl.pallas_call(
        body,
        grid=(G,),
        in_specs=[_bs((NC, BR, F2), _prow), _bs((NC, BR, CW), _prow),
                  _bs((1, F1), _full), _bs((F1, F2), _full)],
        out_specs=_bs((NC, BR, H), _prow),
        out_shape=jax.ShapeDtypeStruct((NC, N, H), jnp.float32),
    )(agg1, nrm, b1r, W2)


def _tc_final(agg2, c_p, nrm, b2r, W3, b3r):
    def body(a_ref, c_ref, n_ref, b2_ref, w_ref, b3_ref, o_ref,
             acca_ref, accb_ref):
        i = pl.program_id(0)

        @pl.when(i == 0)
        def _():
            acca_ref[...] = jnp.zeros((1, H), jnp.float32)
            accb_ref[...] = jnp.zeros((1, H), jnp.float32)

        ns = n_ref[0, :, 0:1]
        nd = n_ref[1, :, 0:1]
        h2a = jnp.maximum(a_ref[0] * nd + b2_ref[:, :H], 0.0)
        h2b = jnp.maximum(a_ref[1] * nd + b2_ref[:, H:], 0.0)
        wv = c_ref[0, :, 0:1] * ns * (1.0 / N)
        acca_ref[...] += jnp.sum(h2a * wv, axis=0, keepdims=True)
        accb_ref[...] += jnp.sum(h2b * wv, axis=0, keepdims=True)

        @pl.when(i == G - 1)
        def _():
            o_ref[...] = (_dot(acca_ref[...], w_ref[:H])
                          + _dot(accb_ref[...], w_ref[H:]) + b3_ref[...])

    return pl.pallas_call(
        body,
        grid=(G,),
        in_specs=[_bs((NC, BR, H), _prow), _bs((NC, BR, CW), _prow),
                  _bs((NC, BR, CW), _prow),
                  _bs((1, F2), _full), _bs((F3, F3), _full),
                  _bs((1, F3), _full)],
        out_specs=_bs((1, F3), _full),
        out_shape=jax.ShapeDtypeStruct((1, F3), jnp.float32),
        scratch_shapes=[pltpu.VMEM((1, H), jnp.float32),
                        pltpu.VMEM((1, H), jnp.float32)],
    )(agg2, c_p, nrm, b2r, W3, b3r)


def kernel(x, edge_index, W1, b1, W2, b2, W3, b3):
    e3 = edge_index.reshape(2, NS, EPT)
    ones16 = jnp.ones((C, CW), jnp.float32)
    z16 = jnp.zeros((RZ, CW), jnp.float32)
    z32 = jnp.zeros((RZ, H), jnp.float32)
    z64 = jnp.zeros((RZ, F2), jnp.float32)

    deg = _sc_degrees(e3, ones16, z16)
    t1pair, nrm = _tc_prep(x, W1, deg)
    agg1 = _sc_layer1(t1pair, e3, z64)
    c_p = _sc_cpass(nrm[1], e3, z16)
    t2pair = _tc_mid(agg1, nrm, b1.reshape(1, F1), W2)
    agg2 = _sc_layer2(t2pair, e3, z32)
    return _tc_final(agg2, c_p, nrm, b2.reshape(1, F2), W3, b3.reshape(1, F3))


# speedup vs baseline: 1.1359x; 1.1359x over previous
import functools

import jax
import jax.numpy as jnp
from jax import lax
from jax.experimental import pallas as pl
from jax.experimental.pallas import tpu as pltpu
from jax.experimental.pallas import tpu_sc as plsc

N = 10000
E = 320000
NC, NS = 2, 16
EPT = E // NS
C = 80
NCHUNK = EPT // C
RB = 624
REM = N - RB * NS
RZ = RB + REM
F1, F2, F3 = 128, 64, 64
H = F2 // 2
CW = 16

_SC_PARAMS = pltpu.CompilerParams(use_tc_tiling_on_sc=False)


def _mesh():
    return plsc.VectorSubcoreMesh(core_axis_name="c", subcore_axis_name="s")


def _zero_acc(sid, z_h, acc):
    pltpu.sync_copy(z_h.at[pl.ds(0, RB)], acc.at[pl.ds(sid * RB, RB)])

    @pl.when(sid == 0)
    def _():
        pltpu.sync_copy(z_h.at[pl.ds(0, REM)], acc.at[pl.ds(RB * NS, REM)])


def _copy_out(sid, acc, out2d):
    rows = pl.ds(sid * RB, RB)
    pltpu.sync_copy(acc.at[rows], out2d.at[rows])

    @pl.when(sid == 0)
    def _():
        tail = pl.ds(RB * NS, REM)
        pltpu.sync_copy(acc.at[tail], out2d.at[tail])


def _sc_degrees(e3, ones_h, zeros_h):
    @functools.partial(
        pl.kernel,
        out_type=jax.ShapeDtypeStruct((NC, N, CW), jnp.float32),
        mesh=_mesh(),
        compiler_params=_SC_PARAMS,
        scratch_types=[
            pltpu.VMEM((EPT,), jnp.int32),
            pltpu.VMEM((C, CW), jnp.float32),
            pltpu.VMEM_SHARED((N, CW), jnp.float32),
            pltpu.SemaphoreType.DMA,
        ],
    )
    def k(e_h, ones_hr, z_h, deg_h, idx_v, ones_v, acc_s, ssem):
        cid = lax.axis_index("c")
        sid = lax.axis_index("s")
        _zero_acc(sid, z_h, acc_s)
        pltpu.sync_copy(ones_hr, ones_v)
        pltpu.sync_copy(e_h.at[cid].at[sid], idx_v)
        plsc.subcore_barrier()

        @pl.loop(0, NCHUNK, step=5)
        def _(j):
            for o in range(5):
                pltpu.async_copy(
                    ones_v, acc_s.at[idx_v.at[pl.ds((j + o) * C, C)]], ssem,
                    add=True)
            for o in range(5):
                pltpu.make_async_copy(
                    ones_v, acc_s.at[idx_v.at[pl.ds(0, C)]], ssem).wait()

        plsc.subcore_barrier()
        _copy_out(sid, acc_s, deg_h.at[cid])

    return k(e3, ones_h, zeros_h)


def _sc_propagate(tables, e3, zeros_list, Ds, K, swaps, splits,
                  edge_split=False):
    NT = len(tables)
    NCH = NCHUNK // 2 if edge_split else NCHUNK
    NE_ = EPT // 2 if edge_split else EPT
    NB = NCH // K
    assert NCH % K == 0

    bufs_types = [pltpu.VMEM((C, D), jnp.float32)
                  for D in Ds for _k in range(K)]
    acc_types = [pltpu.VMEM_SHARED((N, D), jnp.float32) for D in Ds]
    sem_types = [pltpu.SemaphoreType.DMA] * (2 * NT)

    @functools.partial(
        pl.kernel,
        out_type=tuple(jax.ShapeDtypeStruct((NC, N, D), jnp.float32) for D in Ds),
        mesh=_mesh(),
        compiler_params=_SC_PARAMS,
        scratch_types=[
            pltpu.VMEM((NE_,), jnp.int32),
            pltpu.VMEM((NE_,), jnp.int32),
        ] + bufs_types + acc_types + sem_types,
    )
    def k(*refs):
        t_h = refs[:NT]
        e_h = refs[NT]
        z_h = refs[NT + 1:NT + 1 + NT]
        agg_h = refs[NT + 1 + NT:NT + 1 + 2 * NT]
        src_v, dst_v = refs[3 * NT + 1], refs[3 * NT + 2]
        p = 3 * NT + 3
        bufs = [[refs[p + i * K + k_] for k_ in range(K)] for i in range(NT)]
        p += NT * K
        accs = refs[p:p + NT]
        p += NT
        gsem = refs[p:p + NT]
        ssem = refs[p + NT:p + 2 * NT]

        cid = lax.axis_index("c")
        sid = lax.axis_index("s")
        for i in range(NT):
            _zero_acc(sid, z_h[i], accs[i])
        if edge_split:
            half = pl.ds(cid * NE_, NE_)
            pltpu.sync_copy(e_h.at[0].at[sid].at[half], src_v)
            pltpu.sync_copy(e_h.at[1].at[sid].at[half], dst_v)
        else:
            pltpu.sync_copy(e_h.at[0].at[sid], src_v)
            pltpu.sync_copy(e_h.at[1].at[sid], dst_v)
        plsc.subcore_barrier()

        def table(i):
            return t_h[i].at[cid] if splits[i] else t_h[i]

        def gidx(i, j):
            v = dst_v if swaps[i] else src_v
            return v.at[pl.ds(j * C, C)]

        def sidx(i, j):
            v = src_v if swaps[i] else dst_v
            return v.at[pl.ds(j * C, C)]

        @pl.loop(0, NB)
        def _(b):
            base = b * K
            for k_ in range(K):
                for i in range(NT):
                    pltpu.async_copy(table(i).at[gidx(i, base + k_)],
                                     bufs[i][k_], gsem[i])
            for k_ in range(K):
                for i in range(NT):
                    pltpu.make_async_copy(table(i).at[gidx(i, 0)],
                                          bufs[i][0], gsem[i]).wait()
            for k_ in range(K):
                for i in range(NT):
                    pltpu.async_copy(bufs[i][k_],
                                     accs[i].at[sidx(i, base + k_)],
                                     ssem[i], add=True)
            for k_ in range(K):
                for i in range(NT):
                    pltpu.make_async_copy(bufs[i][0],
                                          accs[i].at[sidx(i, 0)],
                                          ssem[i]).wait()

        plsc.subcore_barrier()
        for i in range(NT):
            _copy_out(sid, accs[i], agg_h[i].at[cid])

    return k(*tables, e3, *zeros_list)


def _sc_layer1(t1pair, e3, z64):
    return _sc_propagate([t1pair], e3, [z64], [F2], 5, [False], [True])[0]


def _sc_cpass(tnd, e3, z16):
    return _sc_propagate([tnd], e3, [z16], [CW], 5, [True], [False],
                         edge_split=True)[0]


def _sc_layer2(t2, e3, z64):
    return _sc_propagate([t2], e3, [z64], [F2], 5, [False], [False],
                         edge_split=True)[0]


G = 10
BR = N // G


def _bs(shape, im):
    return pl.BlockSpec(shape, im)


def _row(i):
    return (i, 0)


def _prow(i):
    return (0, i, 0)


def _full(i):
    return (0, 0)


def _dot(a, b):
    ah = a.astype(jnp.bfloat16)
    al = (a - ah.astype(jnp.float32)).astype(jnp.bfloat16)
    bh = b.astype(jnp.bfloat16)
    bl = (b - bh.astype(jnp.float32)).astype(jnp.bfloat16)

    def d(u, v):
        return lax.dot_general(u, v, (((1,), (0,)), ((), ())),
                               preferred_element_type=jnp.float32)

    return d(ah, bh) + d(ah, bl) + d(al, bh)


def _tc_prep(x, W1, deg):
    def body(x_ref, w_ref, dg_ref, t1p_ref, nrm_ref):
        dego = dg_ref[0, :, 0:1]
        degi = dg_ref[1, :, 0:1]
        ns = jnp.where(dego > 0, lax.rsqrt(dego), 0.0)
        nd = jnp.where(degi > 0, lax.rsqrt(degi), 0.0)
        xs = x_ref[...] * ns
        t1p_ref[0] = _dot(xs, w_ref[:, :F2])
        t1p_ref[1] = _dot(xs, w_ref[:, F2:])
        nrm_ref[0] = jnp.broadcast_to(ns, (BR, CW))
        nrm_ref[1] = jnp.broadcast_to(nd, (BR, CW))

    return pl.pallas_call(
        body,
        grid=(G,),
        in_specs=[_bs((BR, F1), _row), _bs((F1, F1), _full),
                  _bs((NC, BR, CW), _prow)],
        out_specs=(_bs((NC, BR, F2), _prow), _bs((NC, BR, CW), _prow)),
        out_shape=(jax.ShapeDtypeStruct((NC, N, F2), jnp.float32),
                   jax.ShapeDtypeStruct((NC, N, CW), jnp.float32)),
    )(x, W1, deg)


def _tc_mid(agg1, nrm, b1r, W2):
    def body(a_ref, n_ref, b_ref, w_ref, t2_ref):
        ns = n_ref[0, :, 0:1]
        nd = n_ref[1, :, 0:1]
        h1a = jnp.maximum(a_ref[0] * nd + b_ref[:, :F2], 0.0) * ns
        h1b = jnp.maximum(a_ref[1] * nd + b_ref[:, F2:], 0.0) * ns
        t2_ref[...] = _dot(h1a, w_ref[:F2]) + _dot(h1b, w_ref[F2:])

    return pl.pallas_call(
        body,
        grid=(G,),
        in_specs=[_bs((NC, BR, F2), _prow), _bs((NC, BR, CW), _prow),
                  _bs((1, F1), _full), _bs((F1, F2), _full)],
        out_specs=_bs((BR, F2), _row),
        out_shape=jax.ShapeDtypeStruct((N, F2), jnp.float32),
    )(agg1, nrm, b1r, W2)


def _tc_final(agg2, c_p, nrm, b2r, W3, b3r):
    def body(a_ref, c_ref, n_ref, b2_ref, w_ref, b3_ref, o_ref, acc_ref):
        i = pl.program_id(0)

        @pl.when(i == 0)
        def _():
            acc_ref[...] = jnp.zeros((1, F2), jnp.float32)

        ns = n_ref[0, :, 0:1]
        nd = n_ref[1, :, 0:1]
        h2 = jnp.maximum((a_ref[0] + a_ref[1]) * nd + b2_ref[...], 0.0)
        c = c_ref[0, :, 0:1] + c_ref[1, :, 0:1]
        wv = c * ns * (1.0 / N)
        acc_ref[...] += jnp.sum(h2 * wv, axis=0, keepdims=True)

        @pl.when(i == G - 1)
        def _():
            o_ref[...] = _dot(acc_ref[...], w_ref[...]) + b3_ref[...]

    return pl.pallas_call(
        body,
        grid=(G,),
        in_specs=[_bs((NC, BR, F2), _prow), _bs((NC, BR, CW), _prow),
                  _bs((NC, BR, CW), _prow),
                  _bs((1, F2), _full), _bs((F3, F3), _full),
                  _bs((1, F3), _full)],
        out_specs=_bs((1, F3), _full),
        out_shape=jax.ShapeDtypeStruct((1, F3), jnp.float32),
        scratch_shapes=[pltpu.VMEM((1, F2), jnp.float32)],
    )(agg2, c_p, nrm, b2r, W3, b3r)


def kernel(x, edge_index, W1, b1, W2, b2, W3, b3):
    e3 = edge_index.reshape(2, NS, EPT)
    ones16 = jnp.ones((C, CW), jnp.float32)
    z16 = jnp.zeros((RZ, CW), jnp.float32)
    z64 = jnp.zeros((RZ, F2), jnp.float32)

    deg = _sc_degrees(e3, ones16, z16)
    t1pair, nrm = _tc_prep(x, W1, deg)
    agg1 = _sc_layer1(t1pair, e3, z64)
    c_p = _sc_cpass(nrm[1], e3, z16)
    t2 = _tc_mid(agg1, nrm, b1.reshape(1, F1), W2)
    agg2 = _sc_layer2(t2, e3, z64)
    return _tc_final(agg2, c_p, nrm, b2.reshape(1, F2), W3, b3.reshape(1, F3))
